# baseline (speedup 1.0000x reference)
import functools

import jax
import jax.numpy as jnp
from jax import lax
from jax.experimental import pallas as pl
from jax.experimental.pallas import tpu as pltpu
from jax.experimental.pallas import tpu_sc as plsc

_D = 64
_SCALE = 8.0
_NC = 2
_NS = 16
_NW = _NC * _NS
_NBUF = 4
_LEAD = 2


def _sc_gather_scale_2d(table, x2d, chunk_rows):
    R, C = x2d.shape
    CR = chunk_rows
    rows_per_w = R // _NW
    n_chunks = rows_per_w // CR
    n_groups = n_chunks // _NBUF

    mesh = plsc.VectorSubcoreMesh(core_axis_name="c", subcore_axis_name="s")

    @functools.partial(
        pl.kernel,
        out_type=jax.ShapeDtypeStruct((R, C, _D), jnp.float32),
        mesh=mesh,
        scratch_types=[
            pltpu.VMEM((rows_per_w, C), jnp.int32),
            *[pltpu.VMEM((CR, C, _D), jnp.float32) for _ in range(_NBUF)],
            *[pltpu.SemaphoreType.DMA for _ in range(2 * _NBUF)],
        ],
        compiler_params=pltpu.CompilerParams(use_tc_tiling_on_sc=False),
    )
    def k(table_hbm, x_hbm, out_hbm, idx_v,
          b0, b1, b2, b3, g0, g1, g2, g3, o0, o1, o2, o3):
        bufs = (b0, b1, b2, b3)
        gsems = (g0, g1, g2, g3)
        osems = (o0, o1, o2, o3)
        wid = lax.axis_index("s") * _NC + lax.axis_index("c")
        xrow0 = wid * rows_per_w

        pltpu.sync_copy(x_hbm.at[pl.ds(xrow0, rows_per_w)], idx_v)

        def gather_desc(j, s, r):
            return pltpu.make_async_copy(
                table_hbm.at[idx_v.at[j * CR + r]],
                bufs[s].at[r],
                gsems[s])

        def store_desc(j, s):
            return pltpu.make_async_copy(
                bufs[s],
                out_hbm.at[pl.ds(xrow0 + j * CR, CR)],
                osems[s])

        def start_gather(j, s):
            for r in range(CR):
                gather_desc(j, s, r).start()

        def wait_gather(j, s):
            for r in range(CR):
                gather_desc(j, s, r).wait()

        for j0 in range(_LEAD):
            start_gather(j0, j0)

        def group(gi, carry):
            for s in range(_NBUF):
                j = gi * _NBUF + s
                ns = (s + _LEAD) % _NBUF

                @pl.when(j + _LEAD < n_chunks)
                def _():
                    @pl.when(j >= _NBUF - _LEAD)
                    def _():
                        store_desc(j - (_NBUF - _LEAD), ns).wait()
                    start_gather(j + _LEAD, ns)

                wait_gather(j, s)

                def scale_rows(i, c):
                    for cc in range(C):
                        for q in range(_D // 16):
                            sl = (i, cc, pl.ds(q * 16, 16))
                            bufs[s][sl] = bufs[s][sl] * _SCALE
                    return c

                lax.fori_loop(0, CR, scale_rows, 0)
                store_desc(j, s).start()
            return carry

        lax.fori_loop(0, n_groups, group, 0)

        for m in range(n_chunks - _NBUF, n_chunks):
            store_desc(m, m % _NBUF).wait()

    return k(table, x2d)


def kernel(x, lut_weight):
    xi = x.astype(jnp.int32)
    if (x.ndim == 2 and x.shape[0] % (_NW * 16 * _NBUF) == 0
            and 0 < x.shape[1] <= 32):
        out = _sc_gather_scale_2d(lut_weight, xi, 16)
        return out
    n = x.size
    gran = _NW * 4 * _NBUF * 128
    pad = (-n) % gran
    flat = xi.reshape(-1)
    if pad:
        flat = jnp.concatenate([flat, jnp.zeros((pad,), jnp.int32)])
    out = _sc_gather_scale_2d(lut_weight, flat.reshape(-1, 128), 4)
    return out.reshape(-1, _D)[:n].reshape(*x.shape, _D)
